# baseline (speedup 1.0000x reference)
import functools

import jax
import jax.numpy as jnp
from jax import lax
from jax.experimental import pallas as pl
from jax.experimental.pallas import tpu as pltpu
from jax.experimental.pallas import tpu_sc as plsc

NC = 2
NS = 16
LCH = 128


def _make_agg(n, d, n_pad, e):
  nw = NC * NS
  assert e % LCH == 0
  ch_total = e // LCH
  ch_lo = ch_total // nw
  n_hi = ch_total - ch_lo * nw
  ch_hi = ch_lo + (1 if n_hi else 0)
  chp = (ch_hi + 1) // 2
  rows_per_tile = n_pad // NS

  mesh = plsc.VectorSubcoreMesh(
      core_axis_name="c", subcore_axis_name="s",
      num_cores=NC, num_subcores=NS)

  @functools.partial(
      pl.kernel,
      out_type=jax.ShapeDtypeStruct((NC, n_pad, d), jnp.float32),
      mesh=mesh,
      scratch_types=[
          pltpu.VMEM((2, chp * LCH), jnp.int32),
          pltpu.VMEM((LCH, d), jnp.float32),
          pltpu.VMEM((LCH, d), jnp.float32),
          pltpu.VMEM_SHARED((n_pad, d), jnp.float32),
          pltpu.SemaphoreType.DMA,
          pltpu.SemaphoreType.DMA,
      ],
  )
  def agg_kernel(h_hbm, ei_hbm, out_hbm,
                 idx, buf_a, buf_b, acc, sem_a, sem_b):
    c = lax.axis_index("c")
    s = lax.axis_index("s")

    zv = jnp.zeros((16,), jnp.float32)

    @pl.loop(0, LCH)
    def _(r):
      for k in range(d // 16):
        buf_a[r, pl.ds(k * 16, 16)] = zv

    zbase = s * rows_per_tile
    zfull, zrem = divmod(rows_per_tile, LCH)
    for b in range(zfull):
      pltpu.sync_copy(buf_a, acc.at[pl.ds(zbase + b * LCH, LCH)])
    if zrem:
      pltpu.sync_copy(buf_a.at[pl.ds(0, zrem)],
                      acc.at[pl.ds(zbase + zfull * LCH, zrem)])
    plsc.subcore_barrier()

    r_c = n_hi // NC + jnp.where(c < n_hi % NC, 1, 0)
    core_base = c * NS * ch_lo + c * (n_hi // NC) + jnp.minimum(c, n_hi % NC)
    extra = s < r_c
    start = pl.multiple_of(
        (core_base
         + jnp.where(extra, s * ch_hi, r_c * ch_hi + (s - r_c) * ch_lo))
        * LCH, LCH)

    def stage(phase_start, nrows):
      pltpu.sync_copy(ei_hbm.at[:, pl.ds(start + phase_start * LCH,
                                         nrows * LCH)],
                      idx.at[:, pl.ds(0, nrows * LCH)])

    def gather(j, buf, sem):
      half = LCH // 2
      pltpu.async_copy(h_hbm.at[idx.at[0, pl.ds(j * LCH, half)]],
                       buf.at[pl.ds(0, half)], sem)
      pltpu.async_copy(h_hbm.at[idx.at[0, pl.ds(j * LCH + half, half)]],
                       buf.at[pl.ds(half, half)], sem)

    def wait(buf, sem):
      pltpu.make_async_copy(h_hbm.at[pl.ds(0, LCH)], buf, sem).wait()

    def scatter_add(j, buf):
      pltpu.sync_copy(buf, acc.at[idx.at[1, pl.ds(j * LCH, LCH)]], add=True)

    def run_phase(nch):
      gather(0, buf_a, sem_a)

      @pl.loop(0, nch, step=2)
      def _(g):
        gather(g + 1, buf_b, sem_b)
        wait(buf_a, sem_a)
        scatter_add(g, buf_a)

        @pl.when(g + 2 < nch)
        def _():
          gather(g + 2, buf_a, sem_a)

        wait(buf_b, sem_b)
        scatter_add(g + 1, buf_b)

    def run_tail(j):
      gather(j, buf_a, sem_a)
      wait(buf_a, sem_a)
      scatter_add(j, buf_a)

    assert chp % 2 == 0 and ch_lo >= chp
    stage(0, chp)
    run_phase(chp)

    r_lo, r_hi = ch_lo - chp, ch_hi - chp

    @pl.when(extra)
    def _():
      stage(chp, r_hi)
      run_phase(r_hi - (r_hi % 2))
      if r_hi % 2:
        run_tail(r_hi - 1)

    @pl.when(jnp.logical_not(extra))
    def _():
      if r_lo:
        stage(chp, r_lo)
        run_phase(r_lo - (r_lo % 2))
        if r_lo % 2:
          run_tail(r_lo - 1)

    plsc.subcore_barrier()
    pltpu.sync_copy(acc.at[pl.ds(s * rows_per_tile, rows_per_tile)],
                    out_hbm.at[c, pl.ds(s * rows_per_tile, rows_per_tile)])

  return agg_kernel


def _mlp1_body(x_ref, p_ref, w1_ref, b1_ref, w2_ref, b2_ref, o_ref):
  z = x_ref[...] + p_ref[0] + p_ref[1]
  t = jnp.dot(z, w1_ref[...], preferred_element_type=jnp.float32)
  t = jnp.maximum(t + b1_ref[...], 0.0)
  h = jnp.dot(t, w2_ref[...], preferred_element_type=jnp.float32)
  o_ref[...] = jnp.maximum(h + b2_ref[...], 0.0)


def _mlp2_body(h_ref, q_ref, w1_ref, b1_ref, w2_ref, b2_ref,
               wc_ref, bc_ref, o_ref):
  z = h_ref[...] + q_ref[0] + q_ref[1]
  t = jnp.dot(z, w1_ref[...], preferred_element_type=jnp.float32)
  t = jnp.maximum(t + b1_ref[...], 0.0)
  h2 = jnp.dot(t, w2_ref[...], preferred_element_type=jnp.float32)
  h2 = jnp.maximum(h2 + b2_ref[...], 0.0)
  o = jnp.dot(h2, wc_ref[...], preferred_element_type=jnp.float32)
  o_ref[...] = o + bc_ref[...]


def _full_spec(shape):
  return pl.BlockSpec(shape, lambda i: (0,) * len(shape))


def _mlp1_call(x, p, w1, b1, w2, b2, bm):
  n, d = x.shape
  h = w1.shape[1]
  grid = (n // bm,)
  return pl.pallas_call(
      _mlp1_body,
      grid=grid,
      in_specs=[
          pl.BlockSpec((bm, d), lambda i: (i, 0)),
          pl.BlockSpec((NC, bm, d), lambda i: (0, i, 0)),
          _full_spec(w1.shape),
          _full_spec((1, h)),
          _full_spec(w2.shape),
          _full_spec((1, h)),
      ],
      out_specs=pl.BlockSpec((bm, h), lambda i: (i, 0)),
      out_shape=jax.ShapeDtypeStruct((n, h), jnp.float32),
  )(x, p, w1, b1.reshape(1, -1), w2, b2.reshape(1, -1))


def _mlp2_call(hh, q, w1, b1, w2, b2, wc, bc, bm):
  n, d = hh.shape
  h = w1.shape[1]
  c = wc.shape[1]
  grid = (n // bm,)
  return pl.pallas_call(
      _mlp2_body,
      grid=grid,
      in_specs=[
          pl.BlockSpec((bm, d), lambda i: (i, 0)),
          pl.BlockSpec((NC, bm, d), lambda i: (0, i, 0)),
          _full_spec(w1.shape),
          _full_spec((1, h)),
          _full_spec(w2.shape),
          _full_spec((1, h)),
          _full_spec(wc.shape),
          _full_spec((1, c)),
      ],
      out_specs=pl.BlockSpec((bm, c), lambda i: (i, 0)),
      out_shape=jax.ShapeDtypeStruct((n, c), jnp.float32),
  )(hh, q, w1, b1.reshape(1, -1), w2, b2.reshape(1, -1),
    wc, bc.reshape(1, -1))


def kernel(x, edge_index, W11, b11, W12, b12, W21, b21, W22, b22, Wc, bc):
  n, d = x.shape
  e = edge_index.shape[1]
  n_pad = -(-n // (NS * 8)) * (NS * 8)

  ei = edge_index.astype(jnp.int32)
  padded = bool(e % LCH)
  if padded:
    pad = LCH - e % LCH
    ei = jnp.concatenate(
        [ei, jnp.stack([jnp.full((pad,), n, jnp.int32),
                        jnp.arange(pad, dtype=jnp.int32) * 523 % n])],
        axis=1)
    e = e + pad

  def table(t):
    if padded:
      return jnp.concatenate([t, jnp.zeros((1, d), t.dtype)], axis=0)
    return t

  agg = _make_agg(n, d, n_pad, e)
  bm = 5000 if n % 5000 == 0 else 2000

  p1 = agg(table(x), ei)
  h1 = _mlp1_call(x, p1, W11, b11, W12, b12, bm)
  p2 = agg(table(h1), ei)
  return _mlp2_call(h1, p2, W21, b21, W22, b22, Wc, bc, bm)
